# baseline (speedup 1.0000x reference)
import functools

import jax
import jax.numpy as jnp
import numpy as np
from jax import lax
from jax.experimental import pallas as pl
from jax.experimental.pallas import tpu as pltpu
from jax.experimental.pallas import tpu_sc as plsc

B = 16384
F = 27
V = 100000
D = 16
NF = F - 1
EPS = 1e-5
K = B * NF

_NC, _NS = 2, 16
_NW = _NC * _NS
_PER_W = K // _NW
_N_CHUNK = 8
_CHUNK = _PER_W // _N_CHUNK


@functools.lru_cache(maxsize=None)
def _make_sc_gather():
    mesh = plsc.VectorSubcoreMesh(core_axis_name="c", subcore_axis_name="s")

    @functools.partial(
        pl.kernel,
        mesh=mesh,
        out_type=jax.ShapeDtypeStruct((K, D), jnp.bfloat16),
        scratch_types=[
            pltpu.VMEM((_CHUNK,), jnp.int32),
            pltpu.VMEM((_CHUNK, D), jnp.bfloat16),
            pltpu.SemaphoreType.DMA,
        ],
        compiler_params=pltpu.CompilerParams(use_tc_tiling_on_sc=False),
    )
    def gather_k(idx_hbm, table_hbm, out_hbm, idx_v, rows_v, sem):
        wid = lax.axis_index("s") * _NC + lax.axis_index("c")
        base = wid * _PER_W
        for c in range(_N_CHUNK):
            off = base + c * _CHUNK
            pltpu.sync_copy(idx_hbm.at[pl.ds(off, _CHUNK)], idx_v)
            pltpu.async_copy(table_hbm.at[idx_v], rows_v, sem).wait()
            pltpu.sync_copy(rows_v, out_hbm.at[pl.ds(off, _CHUNK)])

    return gather_k


_TSEG = 256
_TCOL = 8 * _TSEG
_TSUB = 8
_TROWS = _TCOL * _TSUB
_NTB = -(-(F * V) // _TROWS)
_VPAD = _NTB * _TROWS


def _tp_body(tt_ref, out_ref):
    x = tt_ref[...]
    for s in range(_TSUB):
        xs = x[:, s * _TCOL:(s + 1) * _TCOL]
        xden = jnp.concatenate(
            [xs[:, b * _TSEG:(b + 1) * _TSEG] for b in range(8)], axis=0)
        out_ref[pl.ds(s * _TSEG, _TSEG), :] = xden.T.astype(jnp.bfloat16)


def _pack_table(tt, interpret=False):
    return pl.pallas_call(
        _tp_body,
        grid=(_NTB,),
        in_specs=[pl.BlockSpec((D, _TROWS), lambda j: (0, j))],
        out_specs=pl.BlockSpec((_TSUB * _TSEG, 128), lambda j: (j, 0)),
        out_shape=jax.ShapeDtypeStruct((_VPAD // 8, 128), jnp.bfloat16),
        interpret=interpret,
    )(tt)


def _perm(r):
    return 8 * (_TSEG * (r // _TCOL) + (r % _TSEG)) + (r % _TCOL) // _TSEG


_BB = 2048
_RS = float(1.0 / np.sqrt(1.0 + EPS))


def _tc_body(emb_ref, smat_ref, w1_ref, b1_ref, g1_ref, be1_ref,
             w2_ref, b2_ref, g2_ref, be2_ref, w3_ref, b3_ref, g3_ref,
             be3_ref, wout_ref, bout_ref, out_ref):
    e = emb_ref[...].astype(jnp.float32)
    s = jnp.dot(e, smat_ref[...], preferred_element_type=jnp.float32)
    sumsq = jnp.sum(e * e, axis=1, keepdims=True)
    fm = 0.5 * (jnp.sum(s * s, axis=1, keepdims=True) - sumsq)
    h = e
    for w_ref, b_ref, g_ref, be_ref in (
        (w1_ref, b1_ref, g1_ref, be1_ref),
        (w2_ref, b2_ref, g2_ref, be2_ref),
        (w3_ref, b3_ref, g3_ref, be3_ref),
    ):
        h = jnp.dot(h, w_ref[...], preferred_element_type=jnp.float32) + b_ref[...]
        h = jnp.maximum(g_ref[...] * h * _RS + be_ref[...], 0.0)
    mlp = jnp.dot(h, wout_ref[...], preferred_element_type=jnp.float32) + bout_ref[...]
    out_ref[...] = fm * 1.2737 + mlp * 1.341


def _full(shape):
    nd = len(shape)
    return pl.BlockSpec(shape, lambda i: (0,) * nd)


def _tc_head(emb, smat, w1, b1, g1, be1, w2, b2, g2, be2, w3, b3, g3, be3,
             wout, bout, interpret=False):
    grid = (B // _BB,)
    return pl.pallas_call(
        _tc_body,
        grid=grid,
        in_specs=[
            pl.BlockSpec((_BB, NF * D), lambda i: (i, 0)),
            _full(smat.shape), _full(w1.shape), _full(b1.shape),
            _full(g1.shape), _full(be1.shape), _full(w2.shape),
            _full(b2.shape), _full(g2.shape), _full(be2.shape),
            _full(w3.shape), _full(b3.shape), _full(g3.shape),
            _full(be3.shape), _full(wout.shape), _full(bout.shape),
        ],
        out_specs=pl.BlockSpec((_BB, 1), lambda i: (i, 0)),
        out_shape=jax.ShapeDtypeStruct((B, 1), jnp.float32),
        interpret=interpret,
    )(emb, smat, w1, b1, g1, be1, w2, b2, g2, be2, w3, b3, g3, be3,
      wout, bout)


def kernel(x, table, W1, b1, g1, be1, W2, b2, g2, be2, W3, b3, g3, be3,
           Wout, bout):
    offsets = jnp.arange(NF, dtype=x.dtype) * V
    idx = _perm((x[:, :NF] + offsets[None, :]).reshape(-1))
    table_lin = _pack_table(table.T).reshape(_VPAD, D)
    emb = _make_sc_gather()(idx, table_lin).reshape(B, NF * D)
    smat = jnp.tile(jnp.eye(D, dtype=jnp.float32), (NF, 1))
    r2 = lambda v: v.reshape(1, -1)
    return _tc_head(emb, smat, W1, r2(b1), r2(g1), r2(be1), W2, r2(b2),
                    r2(g2), r2(be2), W3, r2(b3), r2(g3), r2(be3), Wout,
                    r2(bout))


# speedup vs baseline: 1.8651x; 1.8651x over previous
import functools

import jax
import jax.numpy as jnp
import numpy as np
from jax import lax
from jax.experimental import pallas as pl
from jax.experimental.pallas import tpu as pltpu
from jax.experimental.pallas import tpu_sc as plsc

B = 16384
F = 27
V = 100000
D = 16
NF = F - 1
EPS = 1e-5
K = B * NF

_NC, _NS = 2, 16
_NW = _NC * _NS
_PER_W = K // _NW
_N_CHUNK = 8
_CHUNK = _PER_W // _N_CHUNK


@functools.lru_cache(maxsize=None)
def _make_sc_gather():
    mesh = plsc.VectorSubcoreMesh(core_axis_name="c", subcore_axis_name="s")

    @functools.partial(
        pl.kernel,
        mesh=mesh,
        out_type=jax.ShapeDtypeStruct((K, D), jnp.float32),
        scratch_types=[
            pltpu.VMEM((_CHUNK,), jnp.int32),
            pltpu.VMEM((_CHUNK, D), jnp.float32),
            pltpu.SemaphoreType.DMA,
        ],
        compiler_params=pltpu.CompilerParams(use_tc_tiling_on_sc=False),
    )
    def gather_k(idx_hbm, table_hbm, out_hbm, idx_v, rows_v, sem):
        wid = lax.axis_index("s") * _NC + lax.axis_index("c")
        base = wid * _PER_W
        for c in range(_N_CHUNK):
            off = base + c * _CHUNK
            pltpu.sync_copy(idx_hbm.at[pl.ds(off, _CHUNK)], idx_v)
            pltpu.async_copy(table_hbm.at[idx_v], rows_v, sem).wait()
            pltpu.sync_copy(rows_v, out_hbm.at[pl.ds(off, _CHUNK)])

    return gather_k


_TSEG = 256
_TCOL = 8 * _TSEG
_TSUB = 8
_TROWS = _TCOL * _TSUB
_NTB = -(-(F * V) // _TROWS)
_VPAD = _NTB * _TROWS


def _tp_body(tt_ref, out_ref):
    x = tt_ref[...]
    for s in range(_TSUB):
        xs = x[:, s * _TCOL:(s + 1) * _TCOL]
        xden = jnp.concatenate(
            [xs[:, b * _TSEG:(b + 1) * _TSEG] for b in range(8)], axis=0)
        out_ref[pl.ds(s * _TSEG, _TSEG), :] = xden.T


def _pack_table(tt, interpret=False):
    return pl.pallas_call(
        _tp_body,
        grid=(_NTB,),
        in_specs=[pl.BlockSpec((D, _TROWS), lambda j: (0, j))],
        out_specs=pl.BlockSpec((_TSUB * _TSEG, 128), lambda j: (j, 0)),
        out_shape=jax.ShapeDtypeStruct((_VPAD // 8, 128), jnp.float32),
        interpret=interpret,
    )(tt)


def _perm(r):
    return 8 * (_TSEG * (r // _TCOL) + (r % _TSEG)) + (r % _TCOL) // _TSEG


_BB = 2048
_RS = float(1.0 / np.sqrt(1.0 + EPS))


def _tc_body(emb_ref, smat_ref, w1_ref, b1_ref, g1_ref, be1_ref,
             w2_ref, b2_ref, g2_ref, be2_ref, w3_ref, b3_ref, g3_ref,
             be3_ref, wout_ref, bout_ref, out_ref):
    e = emb_ref[...]
    s = jnp.dot(e, smat_ref[...], preferred_element_type=jnp.float32)
    sumsq = jnp.sum(e * e, axis=1, keepdims=True)
    fm = 0.5 * (jnp.sum(s * s, axis=1, keepdims=True) - sumsq)
    h = e
    for w_ref, b_ref, g_ref, be_ref in (
        (w1_ref, b1_ref, g1_ref, be1_ref),
        (w2_ref, b2_ref, g2_ref, be2_ref),
        (w3_ref, b3_ref, g3_ref, be3_ref),
    ):
        h = jnp.dot(h, w_ref[...], preferred_element_type=jnp.float32) + b_ref[...]
        h = jnp.maximum(g_ref[...] * h * _RS + be_ref[...], 0.0)
    mlp = jnp.dot(h, wout_ref[...], preferred_element_type=jnp.float32) + bout_ref[...]
    out_ref[...] = fm * 1.2737 + mlp * 1.341


def _full(shape):
    nd = len(shape)
    return pl.BlockSpec(shape, lambda i: (0,) * nd)


def _tc_head(emb, smat, w1, b1, g1, be1, w2, b2, g2, be2, w3, b3, g3, be3,
             wout, bout, interpret=False):
    grid = (B // _BB,)
    return pl.pallas_call(
        _tc_body,
        grid=grid,
        in_specs=[
            pl.BlockSpec((_BB, NF * D), lambda i: (i, 0)),
            _full(smat.shape), _full(w1.shape), _full(b1.shape),
            _full(g1.shape), _full(be1.shape), _full(w2.shape),
            _full(b2.shape), _full(g2.shape), _full(be2.shape),
            _full(w3.shape), _full(b3.shape), _full(g3.shape),
            _full(be3.shape), _full(wout.shape), _full(bout.shape),
        ],
        out_specs=pl.BlockSpec((_BB, 1), lambda i: (i, 0)),
        out_shape=jax.ShapeDtypeStruct((B, 1), jnp.float32),
        interpret=interpret,
    )(emb, smat, w1, b1, g1, be1, w2, b2, g2, be2, w3, b3, g3, be3,
      wout, bout)


def kernel(x, table, W1, b1, g1, be1, W2, b2, g2, be2, W3, b3, g3, be3,
           Wout, bout):
    offsets = jnp.arange(NF, dtype=x.dtype) * V
    idx = _perm((x[:, :NF] + offsets[None, :]).reshape(-1))
    table_lin = _pack_table(table.T).reshape(_VPAD, D)
    emb = _make_sc_gather()(idx, table_lin).reshape(B, NF * D)
    smat = jnp.tile(jnp.eye(D, dtype=jnp.float32), (NF, 1))
    r2 = lambda v: v.reshape(1, -1)
    return _tc_head(emb, smat, W1, r2(b1), r2(g1), r2(be1), W2, r2(b2),
                    r2(g2), r2(be2), W3, r2(b3), r2(g3), r2(be3), Wout,
                    r2(bout))
